# Initial kernel scaffold; baseline (speedup 1.0000x reference)
import jax
import jax.numpy as jnp
from jax.experimental import pallas as pl


def kernel(x, edge_index, edge_attr, batch, edge_emb, W0, as0, ad0, b0, g0, be0, W1, as1, ad1, b1, g1, be1, Wg1, bg1, Wg2, bg2, Wc, bc):
    raise NotImplementedError("write your pallas kernel here")


# speedup vs baseline: 14.2974x; 14.2974x over previous
import functools

import jax
import jax.numpy as jnp
from jax import lax
from jax.experimental import pallas as pl
from jax.experimental.pallas import tpu as pltpu
from jax.experimental.pallas import tpu_sc as plsc

N = 10000
E = 320000
H = 128
NUM_GRAPHS = 16
NC = 2
NS = 16
C = 40
CPT = E // (NC * NS * C)
CB = 25
NB = CPT // CB
NP = 10240
RPT = NP // NS
ZR = 16
WPACK = C // 8

_BLK = 1000
_NBLK = N // _BLK


def _dense0_body(x_ref, w_ref, as_ref, ad_ref, h_ref, s16_ref, d16_ref,
                 ini_ref, sw_ref):
    h = jnp.dot(x_ref[...], w_ref[...], preferred_element_type=jnp.float32)
    h_ref[...] = h
    asv = jnp.sum(h * as_ref[...], axis=1, keepdims=True)
    adv = jnp.sum(h * ad_ref[...], axis=1, keepdims=True)
    s16_ref[...] = jnp.broadcast_to(asv, (_BLK, H))
    d16_ref[...] = jnp.broadcast_to(adv, (_BLK, H))
    es = asv + adv
    es = jnp.where(es >= 0, es, 0.2 * es)
    sw = jnp.exp(es)
    ini_ref[...] = h * sw
    sw_ref[...] = sw


def _dense_bn_body(y_ref, s1_ref, s2_ref, g_ref, be_ref, w_ref, as_ref, ad_ref,
                   h_ref, s16_ref, d16_ref, ini_ref, sw_ref):
    y = y_ref[...]
    mu = s1_ref[...] / N
    var = s2_ref[...] / N - mu * mu
    z = g_ref[...] * (y - mu) / jnp.sqrt(var + 1e-5) + be_ref[...]
    z = jnp.maximum(z, 0.0)
    h = jnp.dot(z, w_ref[...], preferred_element_type=jnp.float32)
    h_ref[...] = h
    asv = jnp.sum(h * as_ref[...], axis=1, keepdims=True)
    adv = jnp.sum(h * ad_ref[...], axis=1, keepdims=True)
    s16_ref[...] = jnp.broadcast_to(asv, (_BLK, H))
    d16_ref[...] = jnp.broadcast_to(adv, (_BLK, H))
    es = asv + adv
    es = jnp.where(es >= 0, es, 0.2 * es)
    sw = jnp.exp(es)
    ini_ref[...] = h * sw
    sw_ref[...] = sw


def _row_spec(w):
    return pl.BlockSpec((_BLK, w), lambda i: (i, 0))


def _vec_spec(w):
    return pl.BlockSpec((1, w), lambda i: (0, 0))


def _mat_spec(r, c):
    return pl.BlockSpec((r, c), lambda i: (0, 0))


_dense_outs = (
    jax.ShapeDtypeStruct((N, H), jnp.float32),
    jax.ShapeDtypeStruct((N, H), jnp.float32),
    jax.ShapeDtypeStruct((N, H), jnp.float32),
    jax.ShapeDtypeStruct((N, H), jnp.float32),
    jax.ShapeDtypeStruct((N, 1), jnp.float32),
)
_dense_out_specs = [_row_spec(H), _row_spec(H), _row_spec(H), _row_spec(H),
                    _row_spec(1)]


def _dense0(x, w, as_, ad_):
    return pl.pallas_call(
        _dense0_body,
        grid=(_NBLK,),
        in_specs=[_row_spec(H), _mat_spec(H, H), _vec_spec(H), _vec_spec(H)],
        out_specs=_dense_out_specs,
        out_shape=_dense_outs,
    )(x, w, as_.reshape(1, H), ad_.reshape(1, H))


def _dense_bn(y, s1, s2, g, be, w, as_, ad_):
    return pl.pallas_call(
        _dense_bn_body,
        grid=(_NBLK,),
        in_specs=[_row_spec(H), _vec_spec(H), _vec_spec(H), _vec_spec(H),
                  _vec_spec(H), _mat_spec(H, H), _vec_spec(H), _vec_spec(H)],
        out_specs=_dense_out_specs,
        out_shape=_dense_outs,
    )(y, s1, s2, g.reshape(1, H), be.reshape(1, H), w,
      as_.reshape(1, H), ad_.reshape(1, H))


def _merge_body(n0_ref, n1_ref, d0_ref, d1_ref, ini_ref, sw_ref, b_ref,
                y_ref, s1_ref, s2_ref):
    num = n0_ref[...] + n1_ref[...] + ini_ref[...]
    den = d0_ref[..., 0:1] + d1_ref[..., 0:1] + sw_ref[...]
    y = num / (den + 1e-16) + b_ref[...]
    y_ref[...] = y
    @pl.when(pl.program_id(0) == 0)
    def _init():
        s1_ref[...] = jnp.zeros_like(s1_ref)
        s2_ref[...] = jnp.zeros_like(s2_ref)
    s1_ref[...] += jnp.sum(y, axis=0, keepdims=True)
    s2_ref[...] += jnp.sum(y * y, axis=0, keepdims=True)


def _merge(n0, n1, d0, d1, ini, sw, b):
    return pl.pallas_call(
        _merge_body,
        grid=(_NBLK,),
        in_specs=[_row_spec(H), _row_spec(H), _row_spec(H), _row_spec(H),
                  _row_spec(H), _row_spec(1), _vec_spec(H)],
        out_specs=[_row_spec(H), _vec_spec(H), _vec_spec(H)],
        out_shape=(
            jax.ShapeDtypeStruct((N, H), jnp.float32),
            jax.ShapeDtypeStruct((1, H), jnp.float32),
            jax.ShapeDtypeStruct((1, H), jnp.float32),
        ),
    )(n0, n1, d0, d1, ini, sw, b.reshape(1, H))


def _final_body(y_ref, s1_ref, s2_ref, g_ref, be_ref, bt_ref, wg1_ref, bg1_ref,
                wg2_ref, bg2_ref, wc_ref, bc_ref, out_ref):
    y = y_ref[...]
    mu = s1_ref[...] / N
    var = s2_ref[...] / N - mu * mu
    z = g_ref[...] * (y - mu) / jnp.sqrt(var + 1e-5) + be_ref[...]
    z = jnp.maximum(z, 0.0)
    t = jnp.dot(z, wg1_ref[...], preferred_element_type=jnp.float32)
    t = jnp.maximum(t + bg1_ref[...], 0.0)
    gate = jnp.dot(t, wg2_ref[...], preferred_element_type=jnp.float32)
    gate = gate + bg2_ref[0, 0]
    bt = bt_ref[...]
    msk = bt == lax.broadcasted_iota(jnp.int32, (1, NUM_GRAPHS), 1)
    gm = jnp.where(msk, gate, -1e30)
    m = jnp.max(gm, axis=0, keepdims=True)
    m = jnp.where(m < -1e29, 0.0, m)
    mb = jnp.sum(jnp.where(msk, m, 0.0), axis=1, keepdims=True)
    ex = jnp.exp(gate - mb)
    den = jnp.sum(jnp.where(msk, ex, 0.0), axis=0, keepdims=True)
    denb = jnp.sum(jnp.where(msk, den, 0.0), axis=1, keepdims=True)
    alpha = ex / (denb + 1e-16)
    ma = jnp.where(msk, alpha, 0.0)
    pooled = lax.dot_general(ma, z, (((0,), (0,)), ((), ())),
                             preferred_element_type=jnp.float32)
    logits = jnp.dot(pooled, wc_ref[...], preferred_element_type=jnp.float32)
    logits = logits + bc_ref[...]
    mx = jnp.max(logits, axis=1, keepdims=True)
    lse = mx + jnp.log(jnp.sum(jnp.exp(logits - mx), axis=1, keepdims=True))
    out_ref[...] = logits - lse


def _final(y, s1, s2, g, be, batch, wg1, bg1, wg2, bg2, wc, bc):
    return pl.pallas_call(
        _final_body,
        out_shape=jax.ShapeDtypeStruct((NUM_GRAPHS, 2), jnp.float32),
    )(y, s1, s2, g.reshape(1, H), be.reshape(1, H), batch.reshape(N, 1),
      wg1, bg1.reshape(1, H), wg2, bg2.reshape(1, 1), wc, bc.reshape(1, 2))


def _edge_body(h_hbm, s16_hbm, d16_hbm, src_hbm, dst_hbm, outn_hbm, outd_hbm,
               wcache_hbm, src_v, dst_v, sc_v, dc_v, rw_v, wpk_v, z_v,
               accum, sem):
    ci = lax.axis_index("c")
    si = lax.axis_index("s")
    r0 = pl.multiple_of(si * RPT, 8)

    zero = jnp.zeros((16,), jnp.float32)
    for i in range(ZR):
        for j in range(H // 16):
            z_v[i, pl.ds(j * 16, 16)] = zero

    def zero_range():
        for b in range(RPT // ZR):
            pltpu.sync_copy(z_v, accum.at[pl.ds(r0 + b * ZR, ZR)])

    zero_range()
    plsc.subcore_barrier()

    def chunk_num(c, carry):
        ga = pltpu.async_copy(s16_hbm.at[src_v.at[c % CB]], sc_v, sem)
        gb = pltpu.async_copy(d16_hbm.at[dst_v.at[c % CB]], dc_v, sem)
        gh = pltpu.async_copy(h_hbm.at[src_v.at[c % CB]], rw_v, sem)
        ga.wait()
        gb.wait()
        gh.wait()
        for r in range(C):
            e = sc_v[r, pl.ds(0, 16)] + dc_v[r, pl.ds(0, 16)]
            e = jnp.where(e >= 0.0, e, 0.2 * e)
            w = jnp.exp(e)
            wpk_v[r // 8, pl.ds((r % 8) * 16, 16)] = w
            for j in range(H // 16):
                sl = pl.ds(j * 16, 16)
                rw_v[r, sl] = rw_v[r, sl] * w
        pltpu.sync_copy(rw_v, accum.at[dst_v.at[c % CB]], add=True)
        pltpu.sync_copy(wpk_v, wcache_hbm.at[ci, si, c])
        return carry

    for b in range(NB):
        pltpu.sync_copy(src_hbm.at[ci, si, b], src_v)
        pltpu.sync_copy(dst_hbm.at[ci, si, b], dst_v)
        lax.fori_loop(b * CB, (b + 1) * CB, chunk_num, 0)
    plsc.subcore_barrier()
    pltpu.sync_copy(accum.at[pl.ds(r0, RPT)], outn_hbm.at[ci, pl.ds(r0, RPT)])
    zero_range()
    plsc.subcore_barrier()

    for r in range(C):
        for j in range(H // 16):
            rw_v[r, pl.ds(j * 16, 16)] = zero

    def chunk_den(c, carry):
        pltpu.sync_copy(wcache_hbm.at[ci, si, c], wpk_v)
        for r in range(C):
            rw_v[r, pl.ds(0, 16)] = wpk_v[r // 8, pl.ds((r % 8) * 16, 16)]
        pltpu.sync_copy(rw_v, accum.at[dst_v.at[c % CB]], add=True)
        return carry

    for b in range(NB):
        pltpu.sync_copy(dst_hbm.at[ci, si, b], dst_v)
        lax.fori_loop(b * CB, (b + 1) * CB, chunk_den, 0)
    plsc.subcore_barrier()
    pltpu.sync_copy(accum.at[pl.ds(r0, RPT)], outd_hbm.at[ci, pl.ds(r0, RPT)])


@functools.partial(
    pl.kernel,
    out_type=(
        jax.ShapeDtypeStruct((NC, NP, H), jnp.float32),
        jax.ShapeDtypeStruct((NC, NP, H), jnp.float32),
        jax.ShapeDtypeStruct((NC, NS, CPT, WPACK, H), jnp.float32),
    ),
    mesh=plsc.VectorSubcoreMesh(core_axis_name="c", subcore_axis_name="s"),
    scratch_types=[
        pltpu.VMEM((CB, C), jnp.int32),
        pltpu.VMEM((CB, C), jnp.int32),
        pltpu.VMEM((C, H), jnp.float32),
        pltpu.VMEM((C, H), jnp.float32),
        pltpu.VMEM((C, H), jnp.float32),
        pltpu.VMEM((WPACK, H), jnp.float32),
        pltpu.VMEM((ZR, H), jnp.float32),
        pltpu.VMEM_SHARED((NP, H), jnp.float32),
        pltpu.SemaphoreType.DMA,
    ],
)
def _edge_sc(h_hbm, s16_hbm, d16_hbm, src_hbm, dst_hbm, outn_hbm, outd_hbm,
             wcache_hbm, src_v, dst_v, sc_v, dc_v, rw_v, wpk_v, z_v,
             accum, sem):
    _edge_body(h_hbm, s16_hbm, d16_hbm, src_hbm, dst_hbm, outn_hbm, outd_hbm,
               wcache_hbm, src_v, dst_v, sc_v, dc_v, rw_v, wpk_v, z_v,
               accum, sem)


def kernel(x, edge_index, edge_attr, batch, edge_emb, W0, as0, ad0, b0, g0,
           be0, W1, as1, ad1, b1, g1, be1, Wg1, bg1, Wg2, bg2, Wc, bc):
    srcr = edge_index[0].reshape(NC, NS, NB, CB, C)
    dstr = edge_index[1].reshape(NC, NS, NB, CB, C)

    h0, s160, d160, ini0, sw0 = _dense0(x, W0, as0, ad0)
    rn0, rd0, _ = _edge_sc(h0, s160, d160, srcr, dstr)
    y1, s1, s2 = _merge(rn0[0, :N], rn0[1, :N], rd0[0, :N], rd0[1, :N],
                        ini0, sw0, b0)

    h1, s161, d161, ini1, sw1 = _dense_bn(y1, s1, s2, g0, be0, W1, as1, ad1)
    rn1, rd1, _ = _edge_sc(h1, s161, d161, srcr, dstr)
    y2, t1, t2 = _merge(rn1[0, :N], rn1[1, :N], rd1[0, :N], rd1[1, :N],
                        ini1, sw1, b1)

    return _final(y2, t1, t2, g1, be1, batch, Wg1, bg1, Wg2, bg2, Wc, bc)


# speedup vs baseline: 17.2637x; 1.2075x over previous
import functools

import jax
import jax.numpy as jnp
from jax import lax
from jax.experimental import pallas as pl
from jax.experimental.pallas import tpu as pltpu
from jax.experimental.pallas import tpu_sc as plsc

N = 10000
E = 320000
H = 128
NUM_GRAPHS = 16
NC = 2
NS = 16
C = 20
CPT = E // (NC * NS * C)
CB = 50
NB = CPT // CB
PAIRS = CB // 2
NP = 10240
RPT = NP // NS
ZR = 16
WPACK = (C + 7) // 8

_BLK = 1000
_NBLK = N // _BLK


def _dense0_body(x_ref, w_ref, as_ref, ad_ref, h_ref, s16_ref, d16_ref,
                 ini_ref, sw_ref):
    h = jnp.dot(x_ref[...], w_ref[...], preferred_element_type=jnp.float32)
    h_ref[...] = h
    asv = jnp.sum(h * as_ref[...], axis=1, keepdims=True)
    adv = jnp.sum(h * ad_ref[...], axis=1, keepdims=True)
    s16_ref[...] = jnp.broadcast_to(asv, (_BLK, H))
    d16_ref[...] = jnp.broadcast_to(adv, (_BLK, H))
    es = asv + adv
    es = jnp.where(es >= 0, es, 0.2 * es)
    sw = jnp.exp(es)
    ini_ref[...] = h * sw
    sw_ref[...] = sw


def _dense_bn_body(y_ref, s1_ref, s2_ref, g_ref, be_ref, w_ref, as_ref, ad_ref,
                   h_ref, s16_ref, d16_ref, ini_ref, sw_ref):
    y = y_ref[...]
    mu = s1_ref[...] / N
    var = s2_ref[...] / N - mu * mu
    z = g_ref[...] * (y - mu) / jnp.sqrt(var + 1e-5) + be_ref[...]
    z = jnp.maximum(z, 0.0)
    h = jnp.dot(z, w_ref[...], preferred_element_type=jnp.float32)
    h_ref[...] = h
    asv = jnp.sum(h * as_ref[...], axis=1, keepdims=True)
    adv = jnp.sum(h * ad_ref[...], axis=1, keepdims=True)
    s16_ref[...] = jnp.broadcast_to(asv, (_BLK, H))
    d16_ref[...] = jnp.broadcast_to(adv, (_BLK, H))
    es = asv + adv
    es = jnp.where(es >= 0, es, 0.2 * es)
    sw = jnp.exp(es)
    ini_ref[...] = h * sw
    sw_ref[...] = sw


def _row_spec(w):
    return pl.BlockSpec((_BLK, w), lambda i: (i, 0))


def _vec_spec(w):
    return pl.BlockSpec((1, w), lambda i: (0, 0))


def _mat_spec(r, c):
    return pl.BlockSpec((r, c), lambda i: (0, 0))


_dense_outs = (
    jax.ShapeDtypeStruct((N, H), jnp.float32),
    jax.ShapeDtypeStruct((N, H), jnp.float32),
    jax.ShapeDtypeStruct((N, H), jnp.float32),
    jax.ShapeDtypeStruct((N, H), jnp.float32),
    jax.ShapeDtypeStruct((N, 1), jnp.float32),
)
_dense_out_specs = [_row_spec(H), _row_spec(H), _row_spec(H), _row_spec(H),
                    _row_spec(1)]


def _dense0(x, w, as_, ad_):
    return pl.pallas_call(
        _dense0_body,
        grid=(_NBLK,),
        in_specs=[_row_spec(H), _mat_spec(H, H), _vec_spec(H), _vec_spec(H)],
        out_specs=_dense_out_specs,
        out_shape=_dense_outs,
    )(x, w, as_.reshape(1, H), ad_.reshape(1, H))


def _dense_bn(y, s1, s2, g, be, w, as_, ad_):
    return pl.pallas_call(
        _dense_bn_body,
        grid=(_NBLK,),
        in_specs=[_row_spec(H), _vec_spec(H), _vec_spec(H), _vec_spec(H),
                  _vec_spec(H), _mat_spec(H, H), _vec_spec(H), _vec_spec(H)],
        out_specs=_dense_out_specs,
        out_shape=_dense_outs,
    )(y, s1, s2, g.reshape(1, H), be.reshape(1, H), w,
      as_.reshape(1, H), ad_.reshape(1, H))


def _merge_body(n0_ref, n1_ref, d0_ref, d1_ref, ini_ref, sw_ref, b_ref,
                y_ref, s1_ref, s2_ref):
    num = n0_ref[...] + n1_ref[...] + ini_ref[...]
    den = d0_ref[..., 0:1] + d1_ref[..., 0:1] + sw_ref[...]
    y = num / (den + 1e-16) + b_ref[...]
    y_ref[...] = y
    @pl.when(pl.program_id(0) == 0)
    def _init():
        s1_ref[...] = jnp.zeros_like(s1_ref)
        s2_ref[...] = jnp.zeros_like(s2_ref)
    s1_ref[...] += jnp.sum(y, axis=0, keepdims=True)
    s2_ref[...] += jnp.sum(y * y, axis=0, keepdims=True)


def _merge(n0, n1, d0, d1, ini, sw, b):
    return pl.pallas_call(
        _merge_body,
        grid=(_NBLK,),
        in_specs=[_row_spec(H), _row_spec(H), _row_spec(H), _row_spec(H),
                  _row_spec(H), _row_spec(1), _vec_spec(H)],
        out_specs=[_row_spec(H), _vec_spec(H), _vec_spec(H)],
        out_shape=(
            jax.ShapeDtypeStruct((N, H), jnp.float32),
            jax.ShapeDtypeStruct((1, H), jnp.float32),
            jax.ShapeDtypeStruct((1, H), jnp.float32),
        ),
    )(n0, n1, d0, d1, ini, sw, b.reshape(1, H))


def _final_body(y_ref, s1_ref, s2_ref, g_ref, be_ref, bt_ref, wg1_ref, bg1_ref,
                wg2_ref, bg2_ref, wc_ref, bc_ref, out_ref):
    y = y_ref[...]
    mu = s1_ref[...] / N
    var = s2_ref[...] / N - mu * mu
    z = g_ref[...] * (y - mu) / jnp.sqrt(var + 1e-5) + be_ref[...]
    z = jnp.maximum(z, 0.0)
    t = jnp.dot(z, wg1_ref[...], preferred_element_type=jnp.float32)
    t = jnp.maximum(t + bg1_ref[...], 0.0)
    gate = jnp.dot(t, wg2_ref[...], preferred_element_type=jnp.float32)
    gate = gate + bg2_ref[0, 0]
    bt = bt_ref[...]
    msk = bt == lax.broadcasted_iota(jnp.int32, (1, NUM_GRAPHS), 1)
    gm = jnp.where(msk, gate, -1e30)
    m = jnp.max(gm, axis=0, keepdims=True)
    m = jnp.where(m < -1e29, 0.0, m)
    mb = jnp.sum(jnp.where(msk, m, 0.0), axis=1, keepdims=True)
    ex = jnp.exp(gate - mb)
    den = jnp.sum(jnp.where(msk, ex, 0.0), axis=0, keepdims=True)
    denb = jnp.sum(jnp.where(msk, den, 0.0), axis=1, keepdims=True)
    alpha = ex / (denb + 1e-16)
    ma = jnp.where(msk, alpha, 0.0)
    pooled = lax.dot_general(ma, z, (((0,), (0,)), ((), ())),
                             preferred_element_type=jnp.float32)
    logits = jnp.dot(pooled, wc_ref[...], preferred_element_type=jnp.float32)
    logits = logits + bc_ref[...]
    mx = jnp.max(logits, axis=1, keepdims=True)
    lse = mx + jnp.log(jnp.sum(jnp.exp(logits - mx), axis=1, keepdims=True))
    out_ref[...] = logits - lse


def _final(y, s1, s2, g, be, batch, wg1, bg1, wg2, bg2, wc, bc):
    return pl.pallas_call(
        _final_body,
        out_shape=jax.ShapeDtypeStruct((NUM_GRAPHS, 2), jnp.float32),
    )(y, s1, s2, g.reshape(1, H), be.reshape(1, H), batch.reshape(N, 1),
      wg1, bg1.reshape(1, H), wg2, bg2.reshape(1, 1), wc, bc.reshape(1, 2))


def _edge_body(h_hbm, s16_hbm, d16_hbm, src_hbm, dst_hbm, outn_hbm, outd_hbm,
               wcache_hbm, src_v, dst_v, sc0_v, sc1_v, dc0_v, dc1_v, rw0_v,
               rw1_v, wpk_v, wp0_v, wp1_v, rww_v, z_v, accum, sem0, sem1):
    ci = lax.axis_index("c")
    si = lax.axis_index("s")
    r0 = pl.multiple_of(si * RPT, 8)

    zero = jnp.zeros((16,), jnp.float32)
    for i in range(ZR):
        for j in range(H // 16):
            z_v[i, pl.ds(j * 16, 16)] = zero

    def zero_range():
        for b in range(RPT // ZR):
            pltpu.sync_copy(z_v, accum.at[pl.ds(r0 + b * ZR, ZR)])

    zero_range()
    plsc.subcore_barrier()

    bufs = ((sc0_v, dc0_v, rw0_v, sem0), (sc1_v, dc1_v, rw1_v, sem1))

    def issue_gather(c, scb, dcb, rwb, sg):
        pltpu.async_copy(s16_hbm.at[src_v.at[c]], scb, sg)
        pltpu.async_copy(d16_hbm.at[dst_v.at[c]], dcb, sg)
        pltpu.async_copy(h_hbm.at[src_v.at[c]], rwb, sg)

    for b in range(NB):
        pltpu.sync_copy(src_hbm.at[ci, si, b], src_v)
        pltpu.sync_copy(dst_hbm.at[ci, si, b], dst_v)
        for p, (scb, dcb, rwb, sg) in enumerate(bufs):
            issue_gather(p, scb, dcb, rwb, sg)

        def pair_num(k, carry):
            for p, (scb, dcb, rwb, sg) in enumerate(bufs):
                c = 2 * k + p
                pltpu.make_async_copy(s16_hbm.at[src_v.at[c]], scb, sg).wait()
                pltpu.make_async_copy(d16_hbm.at[dst_v.at[c]], dcb, sg).wait()
                pltpu.make_async_copy(h_hbm.at[src_v.at[c]], rwb, sg).wait()
                for r in range(C):
                    e = scb[r, pl.ds(0, 16)] + dcb[r, pl.ds(0, 16)]
                    e = jnp.where(e >= 0.0, e, 0.2 * e)
                    w = jnp.exp(e)
                    wpk_v[r // 8, pl.ds((r % 8) * 16, 16)] = w
                    for j in range(H // 16):
                        sl = pl.ds(j * 16, 16)
                        rwb[r, sl] = rwb[r, sl] * w
                pltpu.sync_copy(rwb, accum.at[dst_v.at[c]], add=True)
                pltpu.sync_copy(wpk_v, wcache_hbm.at[ci, si, b * CB + c])

                @pl.when(k < PAIRS - 1)
                def _next():
                    issue_gather(c + 2, scb, dcb, rwb, sg)
            return carry

        lax.fori_loop(0, PAIRS, pair_num, 0)
    plsc.subcore_barrier()
    pltpu.sync_copy(accum.at[pl.ds(r0, RPT)], outn_hbm.at[ci, pl.ds(r0, RPT)])
    zero_range()
    plsc.subcore_barrier()

    for r in range(C):
        for j in range(H // 16):
            rww_v[r, pl.ds(j * 16, 16)] = zero

    wbufs = ((wp0_v, sem0), (wp1_v, sem1))
    for b in range(NB):
        pltpu.sync_copy(dst_hbm.at[ci, si, b], dst_v)
        for p, (wpb, sg) in enumerate(wbufs):
            pltpu.async_copy(wcache_hbm.at[ci, si, b * CB + p], wpb, sg)

        def pair_den(k, carry):
            for p, (wpb, sg) in enumerate(wbufs):
                c = 2 * k + p
                pltpu.make_async_copy(wcache_hbm.at[ci, si, b * CB + c],
                                      wpb, sg).wait()
                for r in range(C):
                    rww_v[r, pl.ds(0, 16)] = wpb[r // 8, pl.ds((r % 8) * 16,
                                                               16)]
                pltpu.sync_copy(rww_v, accum.at[dst_v.at[c]], add=True)

                @pl.when(k < PAIRS - 1)
                def _next():
                    pltpu.async_copy(wcache_hbm.at[ci, si, b * CB + c + 2],
                                     wpb, sg)
            return carry

        lax.fori_loop(0, PAIRS, pair_den, 0)
    plsc.subcore_barrier()
    pltpu.sync_copy(accum.at[pl.ds(r0, RPT)], outd_hbm.at[ci, pl.ds(r0, RPT)])


@functools.partial(
    pl.kernel,
    out_type=(
        jax.ShapeDtypeStruct((NC, NP, H), jnp.float32),
        jax.ShapeDtypeStruct((NC, NP, H), jnp.float32),
        jax.ShapeDtypeStruct((NC, NS, CPT, WPACK, H), jnp.float32),
    ),
    mesh=plsc.VectorSubcoreMesh(core_axis_name="c", subcore_axis_name="s"),
    scratch_types=[
        pltpu.VMEM((CB, C), jnp.int32),
        pltpu.VMEM((CB, C), jnp.int32),
        pltpu.VMEM((C, H), jnp.float32),
        pltpu.VMEM((C, H), jnp.float32),
        pltpu.VMEM((C, H), jnp.float32),
        pltpu.VMEM((C, H), jnp.float32),
        pltpu.VMEM((C, H), jnp.float32),
        pltpu.VMEM((C, H), jnp.float32),
        pltpu.VMEM((WPACK, H), jnp.float32),
        pltpu.VMEM((WPACK, H), jnp.float32),
        pltpu.VMEM((WPACK, H), jnp.float32),
        pltpu.VMEM((C, H), jnp.float32),
        pltpu.VMEM((ZR, H), jnp.float32),
        pltpu.VMEM_SHARED((NP, H), jnp.float32),
        pltpu.SemaphoreType.DMA,
        pltpu.SemaphoreType.DMA,
    ],
)
def _edge_sc(h_hbm, s16_hbm, d16_hbm, src_hbm, dst_hbm, outn_hbm, outd_hbm,
             wcache_hbm, src_v, dst_v, sc0_v, sc1_v, dc0_v, dc1_v, rw0_v,
             rw1_v, wpk_v, wp0_v, wp1_v, rww_v, z_v, accum, sem0, sem1):
    _edge_body(h_hbm, s16_hbm, d16_hbm, src_hbm, dst_hbm, outn_hbm, outd_hbm,
               wcache_hbm, src_v, dst_v, sc0_v, sc1_v, dc0_v, dc1_v, rw0_v,
               rw1_v, wpk_v, wp0_v, wp1_v, rww_v, z_v, accum, sem0, sem1)


def kernel(x, edge_index, edge_attr, batch, edge_emb, W0, as0, ad0, b0, g0,
           be0, W1, as1, ad1, b1, g1, be1, Wg1, bg1, Wg2, bg2, Wc, bc):
    srcr = edge_index[0].reshape(NC, NS, NB, CB, C)
    dstr = edge_index[1].reshape(NC, NS, NB, CB, C)

    h0, s160, d160, ini0, sw0 = _dense0(x, W0, as0, ad0)
    rn0, rd0, _ = _edge_sc(h0, s160, d160, srcr, dstr)
    y1, s1, s2 = _merge(rn0[0, :N], rn0[1, :N], rd0[0, :N], rd0[1, :N],
                        ini0, sw0, b0)

    h1, s161, d161, ini1, sw1 = _dense_bn(y1, s1, s2, g0, be0, W1, as1, ad1)
    rn1, rd1, _ = _edge_sc(h1, s161, d161, srcr, dstr)
    y2, t1, t2 = _merge(rn1[0, :N], rn1[1, :N], rd1[0, :N], rd1[1, :N],
                        ini1, sw1, b1)

    return _final(y2, t1, t2, g1, be1, batch, Wg1, bg1, Wg2, bg2, Wc, bc)
